# baseline (speedup 1.0000x reference)
import functools

import jax
import jax.numpy as jnp
from jax import lax
from jax.experimental import pallas as pl
from jax.experimental.pallas import tpu as pltpu
from jax.experimental.pallas import tpu_sc as plsc

NUM_SEGMENTS = 10000
N_ATOMS = 320000
D_FEAT = 128
HIDDEN = 256
N_OUT = 128

NC = 2
NS = 16
NW = NC * NS
SC_ATOMS = 192000
PW = SC_ATOMS // NW
CHUNK = 128
N_CHUNKS = PW // CHUNK
TAIL = PW - N_CHUNKS * CHUNK
SEG_PER_TILE = 624
SEG_TAIL = NUM_SEGMENTS - NS * SEG_PER_TILE


def _sc_body(
    atoms_hbm, mem_hbm, zeros_hbm, out_hbm,
    idx0, idx1, abuf0, abuf1, tidx, tbuf, acc,
    semi0, sema0, semi1, sema1, sems0, sems1,
):
    c = lax.axis_index("c")
    s = lax.axis_index("s")
    w = s * NC + c
    row0 = pl.multiple_of(s * SEG_PER_TILE, 8)
    idx = (idx0, idx1)
    abuf = (abuf0, abuf1)
    semi = (semi0, semi1)
    sema = (sema0, sema1)
    sems = (sems0, sems1)

    def start_load(g, slot):
        base = pl.multiple_of(w * PW + g * CHUNK, 8)
        pltpu.async_copy(mem_hbm.at[pl.ds(base, CHUNK)], idx[slot], semi[slot])
        pltpu.async_copy(atoms_hbm.at[pl.ds(base, CHUNK)], abuf[slot], sema[slot])

    def wait_load(slot):
        pltpu.make_async_copy(mem_hbm.at[pl.ds(0, CHUNK)], idx[slot], semi[slot]).wait()
        pltpu.make_async_copy(atoms_hbm.at[pl.ds(0, CHUNK)], abuf[slot], sema[slot]).wait()

    def start_scatter(slot):
        pltpu.async_copy(abuf[slot], acc.at[idx[slot]], sems[slot], add=True)

    def wait_scatter(slot):
        pltpu.make_async_copy(abuf[slot], acc.at[idx[slot]], sems[slot]).wait()

    start_load(0, 0)
    start_load(1, 1)

    pltpu.sync_copy(zeros_hbm, acc.at[pl.ds(row0, SEG_PER_TILE)])

    @pl.when(s == 0)
    def _zero_tail():
        pltpu.sync_copy(
            zeros_hbm.at[pl.ds(0, SEG_TAIL)],
            acc.at[pl.ds(NS * SEG_PER_TILE, SEG_TAIL)],
        )

    plsc.subcore_barrier()

    def step(t, carry):
        g0 = t * 2
        g1 = t * 2 + 1
        wait_load(0)
        pltpu.sync_copy(abuf0, acc.at[idx0], add=True)

        @pl.when(g0 + 2 < N_CHUNKS)
        def _next0():
            start_load(g0 + 2, 0)

        wait_load(1)
        pltpu.sync_copy(abuf1, acc.at[idx1], add=True)

        @pl.when(g1 + 2 < N_CHUNKS)
        def _next1():
            start_load(g1 + 2, 1)

        return carry

    lax.fori_loop(0, N_CHUNKS // 2, step, 0)

    if N_CHUNKS % 2 == 1:
        wait_load(0)
        pltpu.sync_copy(abuf0, acc.at[idx0], add=True)

    tbase = pl.multiple_of(w * PW + N_CHUNKS * CHUNK, 8)
    pltpu.sync_copy(mem_hbm.at[pl.ds(tbase, TAIL)], tidx)
    pltpu.sync_copy(atoms_hbm.at[pl.ds(tbase, TAIL)], tbuf)
    pltpu.sync_copy(tbuf, acc.at[tidx], add=True)
    plsc.subcore_barrier()

    orow0 = pl.multiple_of(c * NUM_SEGMENTS + s * SEG_PER_TILE, 8)
    pltpu.sync_copy(
        acc.at[pl.ds(row0, SEG_PER_TILE)],
        out_hbm.at[pl.ds(orow0, SEG_PER_TILE)],
    )

    @pl.when(s == 0)
    def _flush_tail():
        otail = pl.multiple_of(c * NUM_SEGMENTS + NS * SEG_PER_TILE, 8)
        pltpu.sync_copy(
            acc.at[pl.ds(NS * SEG_PER_TILE, SEG_TAIL)],
            out_hbm.at[pl.ds(otail, SEG_TAIL)],
        )


@jax.jit
def _sc_segsum(atoms, mem_i32, zeros):
    mesh = plsc.VectorSubcoreMesh(
        core_axis_name="c", subcore_axis_name="s", num_cores=NC, num_subcores=NS
    )
    f = pl.kernel(
        _sc_body,
        out_type=jax.ShapeDtypeStruct((NC * NUM_SEGMENTS, D_FEAT), jnp.float32),
        mesh=mesh,
        scratch_types=[
            pltpu.VMEM((CHUNK,), jnp.int32),
            pltpu.VMEM((CHUNK,), jnp.int32),
            pltpu.VMEM((CHUNK, D_FEAT), jnp.float32),
            pltpu.VMEM((CHUNK, D_FEAT), jnp.float32),
            pltpu.VMEM((TAIL,), jnp.int32),
            pltpu.VMEM((TAIL, D_FEAT), jnp.float32),
            pltpu.VMEM_SHARED((NUM_SEGMENTS, D_FEAT), jnp.float32),
            pltpu.SemaphoreType.DMA,
            pltpu.SemaphoreType.DMA,
            pltpu.SemaphoreType.DMA,
            pltpu.SemaphoreType.DMA,
            pltpu.SemaphoreType.DMA,
            pltpu.SemaphoreType.DMA,
        ],
    )
    return f(atoms, mem_i32, zeros)


WIN = 128
N_WIN = 79
TC_BLK = 512


def _tc_win_body(s_ref, atoms_hbm, mem_hbm, o_ref, abuf, mbuf, sem_a, sem_m):
    v = pl.program_id(0)
    lo = s_ref[v]
    hi = s_ref[v + 1]
    dn = (lo // 128) * 128
    nblocks = (hi - dn + TC_BLK - 1) // TC_BLK

    def body(i, acc):
        base = jnp.minimum(dn + i * TC_BLK, N_ATOMS - TC_BLK)
        base = pl.multiple_of(base, 128)
        pltpu.make_async_copy(
            atoms_hbm.at[pl.ds(base, TC_BLK)], abuf, sem_a
        ).start()
        pltpu.make_async_copy(
            mem_hbm.at[pl.ds(base, TC_BLK)], mbuf, sem_m
        ).start()
        pltpu.make_async_copy(atoms_hbm.at[pl.ds(base, TC_BLK)], abuf, sem_a).wait()
        pltpu.make_async_copy(mem_hbm.at[pl.ds(base, TC_BLK)], mbuf, sem_m).wait()
        gidx = base + jax.lax.broadcasted_iota(jnp.int32, (TC_BLK, 1), 0)
        valid = (gidx >= jnp.maximum(lo, dn + i * TC_BLK)) & (gidx < hi)
        m = mbuf[...].reshape(TC_BLK, 1)
        seg = v * WIN + jax.lax.broadcasted_iota(jnp.int32, (1, WIN), 1)
        oh = jnp.where((m == seg) & valid, 1.0, 0.0).astype(jnp.float32)
        part = jax.lax.dot_general(
            oh, abuf[...],
            dimension_numbers=(((0,), (0,)), ((), ())),
            preferred_element_type=jnp.float32,
            precision=jax.lax.Precision.HIGHEST,
        )
        return acc + part

    acc0 = jnp.zeros((WIN, D_FEAT), jnp.float32)
    o_ref[...] = lax.fori_loop(0, nblocks, body, acc0)


@jax.jit
def _tc_win_segsum(atoms, mem_i32, starts):
    return pl.pallas_call(
        _tc_win_body,
        grid=(N_WIN,),
        in_specs=[
            pl.BlockSpec(memory_space=pltpu.MemorySpace.SMEM),
            pl.BlockSpec(memory_space=pltpu.MemorySpace.HBM),
            pl.BlockSpec(memory_space=pltpu.MemorySpace.HBM),
        ],
        out_specs=pl.BlockSpec((WIN, D_FEAT), lambda v: (v, 0)),
        out_shape=jax.ShapeDtypeStruct((N_WIN * WIN, D_FEAT), jnp.float32),
        scratch_shapes=[
            pltpu.VMEM((TC_BLK, D_FEAT), jnp.float32),
            pltpu.VMEM((TC_BLK,), jnp.int32),
            pltpu.SemaphoreType.DMA,
            pltpu.SemaphoreType.DMA,
        ],
    )(starts, atoms, mem_i32)


def _mlp_body(p_ref, q_ref, r_ref, w0_ref, b0_ref, w1_ref, b1_ref, o_ref):
    g = p_ref[...] + q_ref[...] + r_ref[...]
    h = jnp.dot(g, w0_ref[...], preferred_element_type=jnp.float32) + b0_ref[...]
    h = jnp.maximum(h, 0.0)
    o = jnp.dot(h, w1_ref[...], preferred_element_type=jnp.float32) + b1_ref[...]
    o_ref[...] = jnp.maximum(o, 0.0)


ROWS = 1000


@jax.jit
def _mlp(partials, tc_partial, W0, b0, W1, b1):
    grid = (NUM_SEGMENTS // ROWS,)
    return pl.pallas_call(
        _mlp_body,
        grid=grid,
        in_specs=[
            pl.BlockSpec((ROWS, D_FEAT), lambda i: (i, 0)),
            pl.BlockSpec((ROWS, D_FEAT), lambda i: (i + NUM_SEGMENTS // ROWS, 0)),
            pl.BlockSpec((ROWS, D_FEAT), lambda i: (i, 0)),
            pl.BlockSpec((D_FEAT, HIDDEN), lambda i: (0, 0)),
            pl.BlockSpec((1, HIDDEN), lambda i: (0, 0)),
            pl.BlockSpec((HIDDEN, N_OUT), lambda i: (0, 0)),
            pl.BlockSpec((1, N_OUT), lambda i: (0, 0)),
        ],
        out_specs=pl.BlockSpec((ROWS, N_OUT), lambda i: (i, 0)),
        out_shape=jax.ShapeDtypeStruct((NUM_SEGMENTS, N_OUT), jnp.float32),
    )(partials, partials, tc_partial, W0, b0, W1, b1)


def kernel(atom_features, membership, W0, b0, W1, b1):
    mem_i32 = membership.astype(jnp.int32)
    zeros = jnp.zeros((SEG_PER_TILE, D_FEAT), jnp.float32)
    starts = jnp.searchsorted(
        mem_i32, jnp.arange(N_WIN + 1, dtype=jnp.int32) * WIN
    ).astype(jnp.int32)
    starts = jnp.maximum(starts, SC_ATOMS)
    partials = _sc_segsum(atom_features, mem_i32, zeros)
    tc_partial = _tc_win_segsum(atom_features, mem_i32, starts)
    return _mlp(partials, tc_partial, W0, b0.reshape(1, HIDDEN), W1, b1.reshape(1, N_OUT))


# speedup vs baseline: 1.7074x; 1.7074x over previous
import functools

import jax
import jax.numpy as jnp
from jax import lax
from jax.experimental import pallas as pl
from jax.experimental.pallas import tpu as pltpu
from jax.experimental.pallas import tpu_sc as plsc

NUM_SEGMENTS = 10000
N_ATOMS = 320000
D_FEAT = 128
HIDDEN = 256
N_OUT = 128

NC = 2
NS = 16
NW = NC * NS
SC_ATOMS = 192000
PW = SC_ATOMS // NW
CHUNK = 128
N_CHUNKS = PW // CHUNK
TAIL = PW - N_CHUNKS * CHUNK
SEG_PER_TILE = 624
SEG_TAIL = NUM_SEGMENTS - NS * SEG_PER_TILE


def _sc_body(
    atoms_hbm, mem_hbm, zeros_hbm, out_hbm,
    idx0, idx1, abuf0, abuf1, tidx, tbuf, acc,
    semi0, sema0, semi1, sema1, sems0, sems1,
):
    c = lax.axis_index("c")
    s = lax.axis_index("s")
    w = s * NC + c
    row0 = pl.multiple_of(s * SEG_PER_TILE, 8)
    idx = (idx0, idx1)
    abuf = (abuf0, abuf1)
    semi = (semi0, semi1)
    sema = (sema0, sema1)
    sems = (sems0, sems1)

    def start_load(g, slot):
        base = pl.multiple_of(w * PW + g * CHUNK, 8)
        pltpu.async_copy(mem_hbm.at[pl.ds(base, CHUNK)], idx[slot], semi[slot])
        pltpu.async_copy(atoms_hbm.at[pl.ds(base, CHUNK)], abuf[slot], sema[slot])

    def wait_load(slot):
        pltpu.make_async_copy(mem_hbm.at[pl.ds(0, CHUNK)], idx[slot], semi[slot]).wait()
        pltpu.make_async_copy(atoms_hbm.at[pl.ds(0, CHUNK)], abuf[slot], sema[slot]).wait()

    def start_scatter(slot):
        pltpu.async_copy(abuf[slot], acc.at[idx[slot]], sems[slot], add=True)

    def wait_scatter(slot):
        pltpu.make_async_copy(abuf[slot], acc.at[idx[slot]], sems[slot]).wait()

    start_load(0, 0)
    start_load(1, 1)

    pltpu.sync_copy(zeros_hbm, acc.at[pl.ds(row0, SEG_PER_TILE)])

    @pl.when(s == 0)
    def _zero_tail():
        pltpu.sync_copy(
            zeros_hbm.at[pl.ds(0, SEG_TAIL)],
            acc.at[pl.ds(NS * SEG_PER_TILE, SEG_TAIL)],
        )

    plsc.subcore_barrier()

    def step(t, carry):
        g0 = t * 2
        g1 = t * 2 + 1
        wait_load(0)
        pltpu.sync_copy(abuf0, acc.at[idx0], add=True)

        @pl.when(g0 + 2 < N_CHUNKS)
        def _next0():
            start_load(g0 + 2, 0)

        wait_load(1)
        pltpu.sync_copy(abuf1, acc.at[idx1], add=True)

        @pl.when(g1 + 2 < N_CHUNKS)
        def _next1():
            start_load(g1 + 2, 1)

        return carry

    lax.fori_loop(0, N_CHUNKS // 2, step, 0)

    if N_CHUNKS % 2 == 1:
        wait_load(0)
        pltpu.sync_copy(abuf0, acc.at[idx0], add=True)

    tbase = pl.multiple_of(w * PW + N_CHUNKS * CHUNK, 8)
    pltpu.sync_copy(mem_hbm.at[pl.ds(tbase, TAIL)], tidx)
    pltpu.sync_copy(atoms_hbm.at[pl.ds(tbase, TAIL)], tbuf)
    pltpu.sync_copy(tbuf, acc.at[tidx], add=True)
    plsc.subcore_barrier()

    orow0 = pl.multiple_of(c * NUM_SEGMENTS + s * SEG_PER_TILE, 8)
    pltpu.sync_copy(
        acc.at[pl.ds(row0, SEG_PER_TILE)],
        out_hbm.at[pl.ds(orow0, SEG_PER_TILE)],
    )

    @pl.when(s == 0)
    def _flush_tail():
        otail = pl.multiple_of(c * NUM_SEGMENTS + NS * SEG_PER_TILE, 8)
        pltpu.sync_copy(
            acc.at[pl.ds(NS * SEG_PER_TILE, SEG_TAIL)],
            out_hbm.at[pl.ds(otail, SEG_TAIL)],
        )


@jax.jit
def _sc_segsum(atoms, mem_i32, zeros):
    mesh = plsc.VectorSubcoreMesh(
        core_axis_name="c", subcore_axis_name="s", num_cores=NC, num_subcores=NS
    )
    f = pl.kernel(
        _sc_body,
        out_type=jax.ShapeDtypeStruct((NC * NUM_SEGMENTS, D_FEAT), jnp.float32),
        mesh=mesh,
        scratch_types=[
            pltpu.VMEM((CHUNK,), jnp.int32),
            pltpu.VMEM((CHUNK,), jnp.int32),
            pltpu.VMEM((CHUNK, D_FEAT), jnp.float32),
            pltpu.VMEM((CHUNK, D_FEAT), jnp.float32),
            pltpu.VMEM((TAIL,), jnp.int32),
            pltpu.VMEM((TAIL, D_FEAT), jnp.float32),
            pltpu.VMEM_SHARED((NUM_SEGMENTS, D_FEAT), jnp.float32),
            pltpu.SemaphoreType.DMA,
            pltpu.SemaphoreType.DMA,
            pltpu.SemaphoreType.DMA,
            pltpu.SemaphoreType.DMA,
            pltpu.SemaphoreType.DMA,
            pltpu.SemaphoreType.DMA,
        ],
    )
    return f(atoms, mem_i32, zeros)


WIN = 128
N_WIN = 79
TC_BLK = 512


def _tc_win_body(s_ref, atoms_hbm, mem_hbm, o_ref, abuf, mbuf, sem_a0, sem_m0, sem_a1, sem_m1):
    v = pl.program_id(0)
    lo = s_ref[v]
    hi = s_ref[v + 1]
    dn = (lo // 128) * 128
    nblocks = (hi - dn + TC_BLK - 1) // TC_BLK
    npairs = (nblocks + 1) // 2
    sem_a = (sem_a0, sem_a1)
    sem_m = (sem_m0, sem_m1)

    def blk_base(b):
        base = jnp.minimum(dn + b * TC_BLK, N_ATOMS - TC_BLK)
        return pl.multiple_of(base, 128)

    def start(b, slot):
        base = blk_base(b)
        pltpu.make_async_copy(
            atoms_hbm.at[pl.ds(base, TC_BLK)], abuf.at[slot], sem_a[slot]
        ).start()
        pltpu.make_async_copy(
            mem_hbm.at[pl.ds(base, TC_BLK)], mbuf.at[slot], sem_m[slot]
        ).start()

    def wait(slot):
        pltpu.make_async_copy(
            atoms_hbm.at[pl.ds(0, TC_BLK)], abuf.at[slot], sem_a[slot]
        ).wait()
        pltpu.make_async_copy(
            mem_hbm.at[pl.ds(0, TC_BLK)], mbuf.at[slot], sem_m[slot]
        ).wait()

    def compute(b, slot, acc):
        base = blk_base(b)
        gidx = base + jax.lax.broadcasted_iota(jnp.int32, (TC_BLK, 1), 0)
        valid = (gidx >= jnp.maximum(lo, dn + b * TC_BLK)) & (gidx < hi)
        m = mbuf[slot].reshape(TC_BLK, 1)
        seg = v * WIN + jax.lax.broadcasted_iota(jnp.int32, (1, WIN), 1)
        oh = jnp.where((m == seg) & valid, 1.0, 0.0).astype(jnp.float32)
        part = jax.lax.dot_general(
            oh, abuf[slot],
            dimension_numbers=(((0,), (0,)), ((), ())),
            preferred_element_type=jnp.float32,
        )
        return acc + part

    @pl.when(nblocks > 0)
    def _p0():
        start(0, 0)

    @pl.when(nblocks > 1)
    def _p1():
        start(1, 1)

    def body(t, acc):
        b0 = t * 2
        b1 = t * 2 + 1
        wait(0)
        acc = compute(b0, 0, acc)

        @pl.when(b0 + 2 < nblocks)
        def _n0():
            start(b0 + 2, 0)

        @pl.when(b1 < nblocks)
        def _w1():
            wait(1)

        acc = compute(b1, 1, acc)

        @pl.when(b1 + 2 < nblocks)
        def _n1():
            start(b1 + 2, 1)

        return acc

    acc0 = jnp.zeros((WIN, D_FEAT), jnp.float32)
    o_ref[...] = lax.fori_loop(0, npairs, body, acc0)


@jax.jit
def _tc_win_segsum(atoms, mem_i32, starts):
    return pl.pallas_call(
        _tc_win_body,
        grid=(N_WIN,),
        in_specs=[
            pl.BlockSpec(memory_space=pltpu.MemorySpace.SMEM),
            pl.BlockSpec(memory_space=pltpu.MemorySpace.HBM),
            pl.BlockSpec(memory_space=pltpu.MemorySpace.HBM),
        ],
        out_specs=pl.BlockSpec((WIN, D_FEAT), lambda v: (v, 0)),
        out_shape=jax.ShapeDtypeStruct((N_WIN * WIN, D_FEAT), jnp.float32),
        scratch_shapes=[
            pltpu.VMEM((2, TC_BLK, D_FEAT), jnp.float32),
            pltpu.VMEM((2, TC_BLK), jnp.int32),
            pltpu.SemaphoreType.DMA,
            pltpu.SemaphoreType.DMA,
            pltpu.SemaphoreType.DMA,
            pltpu.SemaphoreType.DMA,
        ],
    )(starts, atoms, mem_i32)


def _mlp_body(p_ref, q_ref, r_ref, w0_ref, b0_ref, w1_ref, b1_ref, o_ref):
    g = p_ref[...] + q_ref[...] + r_ref[...]
    h = jnp.dot(g, w0_ref[...], preferred_element_type=jnp.float32) + b0_ref[...]
    h = jnp.maximum(h, 0.0)
    o = jnp.dot(h, w1_ref[...], preferred_element_type=jnp.float32) + b1_ref[...]
    o_ref[...] = jnp.maximum(o, 0.0)


ROWS = 1000


@jax.jit
def _mlp(partials, tc_partial, W0, b0, W1, b1):
    grid = (NUM_SEGMENTS // ROWS,)
    return pl.pallas_call(
        _mlp_body,
        grid=grid,
        in_specs=[
            pl.BlockSpec((ROWS, D_FEAT), lambda i: (i, 0)),
            pl.BlockSpec((ROWS, D_FEAT), lambda i: (i + NUM_SEGMENTS // ROWS, 0)),
            pl.BlockSpec((ROWS, D_FEAT), lambda i: (i, 0)),
            pl.BlockSpec((D_FEAT, HIDDEN), lambda i: (0, 0)),
            pl.BlockSpec((1, HIDDEN), lambda i: (0, 0)),
            pl.BlockSpec((HIDDEN, N_OUT), lambda i: (0, 0)),
            pl.BlockSpec((1, N_OUT), lambda i: (0, 0)),
        ],
        out_specs=pl.BlockSpec((ROWS, N_OUT), lambda i: (i, 0)),
        out_shape=jax.ShapeDtypeStruct((NUM_SEGMENTS, N_OUT), jnp.float32),
    )(partials, partials, tc_partial, W0, b0, W1, b1)


def kernel(atom_features, membership, W0, b0, W1, b1):
    mem_i32 = membership.astype(jnp.int32)
    zeros = jnp.zeros((SEG_PER_TILE, D_FEAT), jnp.float32)
    starts = jnp.searchsorted(
        mem_i32, jnp.arange(N_WIN + 1, dtype=jnp.int32) * WIN
    ).astype(jnp.int32)
    starts = jnp.maximum(starts, SC_ATOMS)
    partials = _sc_segsum(atom_features, mem_i32, zeros)
    tc_partial = _tc_win_segsum(atom_features, mem_i32, starts)
    return _mlp(partials, tc_partial, W0, b0.reshape(1, HIDDEN), W1, b1.reshape(1, N_OUT))


# speedup vs baseline: 1.8215x; 1.0668x over previous
import functools

import jax
import jax.numpy as jnp
from jax import lax
from jax.experimental import pallas as pl
from jax.experimental.pallas import tpu as pltpu
from jax.experimental.pallas import tpu_sc as plsc

NUM_SEGMENTS = 10000
N_ATOMS = 320000
D_FEAT = 128
HIDDEN = 256
N_OUT = 128

NC = 2
NS = 16
NW = NC * NS
SC_ATOMS = 192000
PW = SC_ATOMS // NW
CHUNK = 128
N_CHUNKS = PW // CHUNK
TAIL = PW - N_CHUNKS * CHUNK
SEG_PER_TILE = 624
SEG_TAIL = NUM_SEGMENTS - NS * SEG_PER_TILE


def _sc_body(
    atoms_hbm, mem_hbm, zeros_hbm, out_hbm,
    idx0, idx1, abuf0, abuf1, tidx, tbuf, acc,
    semi0, sema0, semi1, sema1, sems0, sems1,
):
    c = lax.axis_index("c")
    s = lax.axis_index("s")
    w = s * NC + c
    row0 = pl.multiple_of(s * SEG_PER_TILE, 8)
    idx = (idx0, idx1)
    abuf = (abuf0, abuf1)
    semi = (semi0, semi1)
    sema = (sema0, sema1)
    sems = (sems0, sems1)

    def start_load(g, slot):
        base = pl.multiple_of(w * PW + g * CHUNK, 8)
        pltpu.async_copy(mem_hbm.at[pl.ds(base, CHUNK)], idx[slot], semi[slot])
        pltpu.async_copy(atoms_hbm.at[pl.ds(base, CHUNK)], abuf[slot], sema[slot])

    def wait_load(slot):
        pltpu.make_async_copy(mem_hbm.at[pl.ds(0, CHUNK)], idx[slot], semi[slot]).wait()
        pltpu.make_async_copy(atoms_hbm.at[pl.ds(0, CHUNK)], abuf[slot], sema[slot]).wait()

    def start_scatter(slot):
        pltpu.async_copy(abuf[slot], acc.at[idx[slot]], sems[slot], add=True)

    def wait_scatter(slot):
        pltpu.make_async_copy(abuf[slot], acc.at[idx[slot]], sems[slot]).wait()

    start_load(0, 0)
    start_load(1, 1)

    pltpu.sync_copy(zeros_hbm, acc.at[pl.ds(row0, SEG_PER_TILE)])

    @pl.when(s == 0)
    def _zero_tail():
        pltpu.sync_copy(
            zeros_hbm.at[pl.ds(0, SEG_TAIL)],
            acc.at[pl.ds(NS * SEG_PER_TILE, SEG_TAIL)],
        )

    plsc.subcore_barrier()

    def step(t, carry):
        g0 = t * 2
        g1 = t * 2 + 1
        wait_load(0)
        pltpu.sync_copy(abuf0, acc.at[idx0], add=True)

        @pl.when(g0 + 2 < N_CHUNKS)
        def _next0():
            start_load(g0 + 2, 0)

        wait_load(1)
        pltpu.sync_copy(abuf1, acc.at[idx1], add=True)

        @pl.when(g1 + 2 < N_CHUNKS)
        def _next1():
            start_load(g1 + 2, 1)

        return carry

    lax.fori_loop(0, N_CHUNKS // 2, step, 0)

    if N_CHUNKS % 2 == 1:
        wait_load(0)
        pltpu.sync_copy(abuf0, acc.at[idx0], add=True)

    tbase = pl.multiple_of(w * PW + N_CHUNKS * CHUNK, 8)
    pltpu.sync_copy(mem_hbm.at[pl.ds(tbase, TAIL)], tidx)
    pltpu.sync_copy(atoms_hbm.at[pl.ds(tbase, TAIL)], tbuf)
    pltpu.sync_copy(tbuf, acc.at[tidx], add=True)
    plsc.subcore_barrier()

    orow0 = pl.multiple_of(c * NUM_SEGMENTS + s * SEG_PER_TILE, 8)
    pltpu.sync_copy(
        acc.at[pl.ds(row0, SEG_PER_TILE)],
        out_hbm.at[pl.ds(orow0, SEG_PER_TILE)],
    )

    @pl.when(s == 0)
    def _flush_tail():
        otail = pl.multiple_of(c * NUM_SEGMENTS + NS * SEG_PER_TILE, 8)
        pltpu.sync_copy(
            acc.at[pl.ds(NS * SEG_PER_TILE, SEG_TAIL)],
            out_hbm.at[pl.ds(otail, SEG_TAIL)],
        )


@jax.jit
def _sc_segsum(atoms, mem_i32, zeros):
    mesh = plsc.VectorSubcoreMesh(
        core_axis_name="c", subcore_axis_name="s", num_cores=NC, num_subcores=NS
    )
    f = pl.kernel(
        _sc_body,
        out_type=jax.ShapeDtypeStruct((NC * NUM_SEGMENTS, D_FEAT), jnp.float32),
        mesh=mesh,
        scratch_types=[
            pltpu.VMEM((CHUNK,), jnp.int32),
            pltpu.VMEM((CHUNK,), jnp.int32),
            pltpu.VMEM((CHUNK, D_FEAT), jnp.float32),
            pltpu.VMEM((CHUNK, D_FEAT), jnp.float32),
            pltpu.VMEM((TAIL,), jnp.int32),
            pltpu.VMEM((TAIL, D_FEAT), jnp.float32),
            pltpu.VMEM_SHARED((NUM_SEGMENTS, D_FEAT), jnp.float32),
            pltpu.SemaphoreType.DMA,
            pltpu.SemaphoreType.DMA,
            pltpu.SemaphoreType.DMA,
            pltpu.SemaphoreType.DMA,
            pltpu.SemaphoreType.DMA,
            pltpu.SemaphoreType.DMA,
        ],
    )
    return f(atoms, mem_i32, zeros)


WIN = 128
N_WIN = 79
TC_BLK = 512


def _tc_win_body(s_ref, atoms_hbm, mem_hbm, o_ref, abuf, mbuf, sem_a0, sem_m0, sem_a1, sem_m1):
    v = pl.program_id(0)
    lo = s_ref[v]
    hi = s_ref[v + 1]
    dn = (lo // 128) * 128
    nblocks = (hi - dn + TC_BLK - 1) // TC_BLK
    npairs = (nblocks + 1) // 2
    sem_a = (sem_a0, sem_a1)
    sem_m = (sem_m0, sem_m1)

    def blk_base(b):
        base = jnp.minimum(dn + b * TC_BLK, N_ATOMS - TC_BLK)
        return pl.multiple_of(base, 128)

    def start(b, slot):
        base = blk_base(b)
        pltpu.make_async_copy(
            atoms_hbm.at[pl.ds(base, TC_BLK)], abuf.at[slot], sem_a[slot]
        ).start()
        pltpu.make_async_copy(
            mem_hbm.at[pl.ds(base, TC_BLK)], mbuf.at[slot], sem_m[slot]
        ).start()

    def wait(slot):
        pltpu.make_async_copy(
            atoms_hbm.at[pl.ds(0, TC_BLK)], abuf.at[slot], sem_a[slot]
        ).wait()
        pltpu.make_async_copy(
            mem_hbm.at[pl.ds(0, TC_BLK)], mbuf.at[slot], sem_m[slot]
        ).wait()

    def compute(b, slot, acc):
        base = blk_base(b)
        rows = TC_BLK // 128
        gidx = (
            base
            + 128 * jax.lax.broadcasted_iota(jnp.int32, (rows, 128), 0)
            + jax.lax.broadcasted_iota(jnp.int32, (rows, 128), 1)
        )
        valid = (gidx >= jnp.maximum(lo, dn + b * TC_BLK)) & (gidx < hi)
        m2 = mbuf[slot].reshape(rows, 128)
        seg = v * WIN + jax.lax.broadcasted_iota(jnp.int32, (WIN, 1, 1), 0)
        oh = jnp.where(
            (m2[None, :, :] == seg) & valid[None, :, :], 1.0, 0.0
        ).astype(jnp.float32).reshape(WIN, TC_BLK)
        part = jax.lax.dot_general(
            oh, abuf[slot],
            dimension_numbers=(((1,), (0,)), ((), ())),
            preferred_element_type=jnp.float32,
        )
        return acc + part

    @pl.when(nblocks > 0)
    def _p0():
        start(0, 0)

    @pl.when(nblocks > 1)
    def _p1():
        start(1, 1)

    def body(t, acc):
        b0 = t * 2
        b1 = t * 2 + 1
        wait(0)
        acc = compute(b0, 0, acc)

        @pl.when(b0 + 2 < nblocks)
        def _n0():
            start(b0 + 2, 0)

        @pl.when(b1 < nblocks)
        def _w1():
            wait(1)

        acc = compute(b1, 1, acc)

        @pl.when(b1 + 2 < nblocks)
        def _n1():
            start(b1 + 2, 1)

        return acc

    acc0 = jnp.zeros((WIN, D_FEAT), jnp.float32)
    o_ref[...] = lax.fori_loop(0, npairs, body, acc0)


@jax.jit
def _tc_win_segsum(atoms, mem_i32, starts):
    return pl.pallas_call(
        _tc_win_body,
        grid=(N_WIN,),
        in_specs=[
            pl.BlockSpec(memory_space=pltpu.MemorySpace.SMEM),
            pl.BlockSpec(memory_space=pltpu.MemorySpace.HBM),
            pl.BlockSpec(memory_space=pltpu.MemorySpace.HBM),
        ],
        out_specs=pl.BlockSpec((WIN, D_FEAT), lambda v: (v, 0)),
        out_shape=jax.ShapeDtypeStruct((N_WIN * WIN, D_FEAT), jnp.float32),
        scratch_shapes=[
            pltpu.VMEM((2, TC_BLK, D_FEAT), jnp.float32),
            pltpu.VMEM((2, TC_BLK), jnp.int32),
            pltpu.SemaphoreType.DMA,
            pltpu.SemaphoreType.DMA,
            pltpu.SemaphoreType.DMA,
            pltpu.SemaphoreType.DMA,
        ],
    )(starts, atoms, mem_i32)


def _mlp_body(p_ref, q_ref, r_ref, w0_ref, b0_ref, w1_ref, b1_ref, o_ref):
    g = p_ref[...] + q_ref[...] + r_ref[...]
    h = jnp.dot(g, w0_ref[...], preferred_element_type=jnp.float32) + b0_ref[...]
    h = jnp.maximum(h, 0.0)
    o = jnp.dot(h, w1_ref[...], preferred_element_type=jnp.float32) + b1_ref[...]
    o_ref[...] = jnp.maximum(o, 0.0)


ROWS = 1000


@jax.jit
def _mlp(partials, tc_partial, W0, b0, W1, b1):
    grid = (NUM_SEGMENTS // ROWS,)
    return pl.pallas_call(
        _mlp_body,
        grid=grid,
        in_specs=[
            pl.BlockSpec((ROWS, D_FEAT), lambda i: (i, 0)),
            pl.BlockSpec((ROWS, D_FEAT), lambda i: (i + NUM_SEGMENTS // ROWS, 0)),
            pl.BlockSpec((ROWS, D_FEAT), lambda i: (i, 0)),
            pl.BlockSpec((D_FEAT, HIDDEN), lambda i: (0, 0)),
            pl.BlockSpec((1, HIDDEN), lambda i: (0, 0)),
            pl.BlockSpec((HIDDEN, N_OUT), lambda i: (0, 0)),
            pl.BlockSpec((1, N_OUT), lambda i: (0, 0)),
        ],
        out_specs=pl.BlockSpec((ROWS, N_OUT), lambda i: (i, 0)),
        out_shape=jax.ShapeDtypeStruct((NUM_SEGMENTS, N_OUT), jnp.float32),
    )(partials, partials, tc_partial, W0, b0, W1, b1)


def kernel(atom_features, membership, W0, b0, W1, b1):
    mem_i32 = membership.astype(jnp.int32)
    zeros = jnp.zeros((SEG_PER_TILE, D_FEAT), jnp.float32)
    starts = jnp.searchsorted(
        mem_i32, jnp.arange(N_WIN + 1, dtype=jnp.int32) * WIN
    ).astype(jnp.int32)
    starts = jnp.maximum(starts, SC_ATOMS)
    partials = _sc_segsum(atom_features, mem_i32, zeros)
    tc_partial = _tc_win_segsum(atom_features, mem_i32, starts)
    return _mlp(partials, tc_partial, W0, b0.reshape(1, HIDDEN), W1, b1.reshape(1, N_OUT))


# speedup vs baseline: 2.3212x; 1.2743x over previous
import functools

import jax
import jax.numpy as jnp
from jax import lax
from jax.experimental import pallas as pl
from jax.experimental.pallas import tpu as pltpu
from jax.experimental.pallas import tpu_sc as plsc

NUM_SEGMENTS = 10000
N_ATOMS = 320000
D_FEAT = 128
HIDDEN = 256
N_OUT = 128

NC = 2
NS = 16
NW = NC * NS
SC_ATOMS = 192000
PW = SC_ATOMS // NW
CHUNK = 128
N_CHUNKS = PW // CHUNK
TAIL = PW - N_CHUNKS * CHUNK
SEG_PER_TILE = 624
SEG_TAIL = NUM_SEGMENTS - NS * SEG_PER_TILE


def _sc_body(
    atoms_hbm, mem_hbm, zeros_hbm, out_hbm,
    idx0, idx1, abuf0, abuf1, tidx, tbuf, acc,
    semi0, sema0, semi1, sema1, sems0, sems1,
):
    c = lax.axis_index("c")
    s = lax.axis_index("s")
    w = s * NC + c
    row0 = pl.multiple_of(s * SEG_PER_TILE, 8)
    idx = (idx0, idx1)
    abuf = (abuf0, abuf1)
    semi = (semi0, semi1)
    sema = (sema0, sema1)
    sems = (sems0, sems1)

    def start_load(g, slot):
        base = pl.multiple_of(w * PW + g * CHUNK, 8)
        pltpu.async_copy(mem_hbm.at[pl.ds(base, CHUNK)], idx[slot], semi[slot])
        pltpu.async_copy(atoms_hbm.at[pl.ds(base, CHUNK)], abuf[slot], sema[slot])

    def wait_load(slot):
        pltpu.make_async_copy(mem_hbm.at[pl.ds(0, CHUNK)], idx[slot], semi[slot]).wait()
        pltpu.make_async_copy(atoms_hbm.at[pl.ds(0, CHUNK)], abuf[slot], sema[slot]).wait()

    def start_scatter(slot):
        pltpu.async_copy(abuf[slot], acc.at[idx[slot]], sems[slot], add=True)

    def wait_scatter(slot):
        pltpu.make_async_copy(abuf[slot], acc.at[idx[slot]], sems[slot]).wait()

    start_load(0, 0)
    start_load(1, 1)

    pltpu.sync_copy(zeros_hbm, acc.at[pl.ds(row0, SEG_PER_TILE)])

    @pl.when(s == 0)
    def _zero_tail():
        pltpu.sync_copy(
            zeros_hbm.at[pl.ds(0, SEG_TAIL)],
            acc.at[pl.ds(NS * SEG_PER_TILE, SEG_TAIL)],
        )

    plsc.subcore_barrier()

    def step(t, carry):
        g0 = t * 2
        g1 = t * 2 + 1
        wait_load(0)
        pltpu.sync_copy(abuf0, acc.at[idx0], add=True)

        @pl.when(g0 + 2 < N_CHUNKS)
        def _next0():
            start_load(g0 + 2, 0)

        wait_load(1)
        pltpu.sync_copy(abuf1, acc.at[idx1], add=True)

        @pl.when(g1 + 2 < N_CHUNKS)
        def _next1():
            start_load(g1 + 2, 1)

        return carry

    lax.fori_loop(0, N_CHUNKS // 2, step, 0)

    if N_CHUNKS % 2 == 1:
        wait_load(0)
        pltpu.sync_copy(abuf0, acc.at[idx0], add=True)

    tbase = pl.multiple_of(w * PW + N_CHUNKS * CHUNK, 8)
    pltpu.sync_copy(mem_hbm.at[pl.ds(tbase, TAIL)], tidx)
    pltpu.sync_copy(atoms_hbm.at[pl.ds(tbase, TAIL)], tbuf)
    pltpu.sync_copy(tbuf, acc.at[tidx], add=True)
    plsc.subcore_barrier()

    orow0 = pl.multiple_of(c * NUM_SEGMENTS + s * SEG_PER_TILE, 8)
    pltpu.sync_copy(
        acc.at[pl.ds(row0, SEG_PER_TILE)],
        out_hbm.at[pl.ds(orow0, SEG_PER_TILE)],
    )

    @pl.when(s == 0)
    def _flush_tail():
        otail = pl.multiple_of(c * NUM_SEGMENTS + NS * SEG_PER_TILE, 8)
        pltpu.sync_copy(
            acc.at[pl.ds(NS * SEG_PER_TILE, SEG_TAIL)],
            out_hbm.at[pl.ds(otail, SEG_TAIL)],
        )


@jax.jit
def _sc_segsum(atoms, mem_i32, zeros):
    mesh = plsc.VectorSubcoreMesh(
        core_axis_name="c", subcore_axis_name="s", num_cores=NC, num_subcores=NS
    )
    f = pl.kernel(
        _sc_body,
        out_type=jax.ShapeDtypeStruct((NC * NUM_SEGMENTS, D_FEAT), jnp.float32),
        mesh=mesh,
        scratch_types=[
            pltpu.VMEM((CHUNK,), jnp.int32),
            pltpu.VMEM((CHUNK,), jnp.int32),
            pltpu.VMEM((CHUNK, D_FEAT), jnp.float32),
            pltpu.VMEM((CHUNK, D_FEAT), jnp.float32),
            pltpu.VMEM((TAIL,), jnp.int32),
            pltpu.VMEM((TAIL, D_FEAT), jnp.float32),
            pltpu.VMEM_SHARED((NUM_SEGMENTS, D_FEAT), jnp.float32),
            pltpu.SemaphoreType.DMA,
            pltpu.SemaphoreType.DMA,
            pltpu.SemaphoreType.DMA,
            pltpu.SemaphoreType.DMA,
            pltpu.SemaphoreType.DMA,
            pltpu.SemaphoreType.DMA,
        ],
    )
    return f(atoms, mem_i32, zeros)


WIN = 128
N_WIN = 79
TC_BLK = 1024


def _tc_win_body(s_ref, atoms_hbm, mem_hbm, o_ref, abuf, mbuf, sem_a0, sem_m0, sem_a1, sem_m1):
    v = pl.program_id(0)
    lo = s_ref[v]
    hi = s_ref[v + 1]
    dn = (lo // 128) * 128
    nblocks = (hi - dn + TC_BLK - 1) // TC_BLK
    npairs = (nblocks + 1) // 2
    sem_a = (sem_a0, sem_a1)
    sem_m = (sem_m0, sem_m1)

    def blk_base(b):
        base = jnp.minimum(dn + b * TC_BLK, N_ATOMS - TC_BLK)
        return pl.multiple_of(base, 128)

    def start(b, slot):
        base = blk_base(b)
        pltpu.make_async_copy(
            atoms_hbm.at[pl.ds(base, TC_BLK)], abuf.at[slot], sem_a[slot]
        ).start()
        pltpu.make_async_copy(
            mem_hbm.at[pl.ds(base, TC_BLK)], mbuf.at[slot], sem_m[slot]
        ).start()

    def wait(slot):
        pltpu.make_async_copy(
            atoms_hbm.at[pl.ds(0, TC_BLK)], abuf.at[slot], sem_a[slot]
        ).wait()
        pltpu.make_async_copy(
            mem_hbm.at[pl.ds(0, TC_BLK)], mbuf.at[slot], sem_m[slot]
        ).wait()

    def compute(b, slot, acc):
        base = blk_base(b)
        rows = TC_BLK // 128
        gidx = (
            base
            + 128 * jax.lax.broadcasted_iota(jnp.int32, (rows, 128), 0)
            + jax.lax.broadcasted_iota(jnp.int32, (rows, 128), 1)
        )
        valid = (gidx >= jnp.maximum(lo, dn + b * TC_BLK)) & (gidx < hi)
        m2 = mbuf[slot].reshape(rows, 128)
        seg = v * WIN + jax.lax.broadcasted_iota(jnp.int32, (WIN, 1, 1), 0)
        oh = jnp.where(
            (m2[None, :, :] == seg) & valid[None, :, :], 1.0, 0.0
        ).astype(jnp.float32).reshape(WIN, TC_BLK)
        part = jax.lax.dot_general(
            oh, abuf[slot],
            dimension_numbers=(((1,), (0,)), ((), ())),
            preferred_element_type=jnp.float32,
        )
        return acc + part

    @pl.when(nblocks > 0)
    def _p0():
        start(0, 0)

    @pl.when(nblocks > 1)
    def _p1():
        start(1, 1)

    def body(t, acc):
        b0 = t * 2
        b1 = t * 2 + 1
        wait(0)
        acc = compute(b0, 0, acc)

        @pl.when(b0 + 2 < nblocks)
        def _n0():
            start(b0 + 2, 0)

        @pl.when(b1 < nblocks)
        def _w1():
            wait(1)

        acc = compute(b1, 1, acc)

        @pl.when(b1 + 2 < nblocks)
        def _n1():
            start(b1 + 2, 1)

        return acc

    acc0 = jnp.zeros((WIN, D_FEAT), jnp.float32)
    o_ref[...] = lax.fori_loop(0, npairs, body, acc0)


@jax.jit
def _tc_win_segsum(atoms, mem_i32, starts):
    return pl.pallas_call(
        _tc_win_body,
        grid=(N_WIN,),
        in_specs=[
            pl.BlockSpec(memory_space=pltpu.MemorySpace.SMEM),
            pl.BlockSpec(memory_space=pltpu.MemorySpace.HBM),
            pl.BlockSpec(memory_space=pltpu.MemorySpace.HBM),
        ],
        out_specs=pl.BlockSpec((WIN, D_FEAT), lambda v: (v, 0)),
        out_shape=jax.ShapeDtypeStruct((N_WIN * WIN, D_FEAT), jnp.float32),
        scratch_shapes=[
            pltpu.VMEM((2, TC_BLK, D_FEAT), jnp.float32),
            pltpu.VMEM((2, TC_BLK), jnp.int32),
            pltpu.SemaphoreType.DMA,
            pltpu.SemaphoreType.DMA,
            pltpu.SemaphoreType.DMA,
            pltpu.SemaphoreType.DMA,
        ],
    )(starts, atoms, mem_i32)


def _mlp_body(p_ref, q_ref, r_ref, w0_ref, b0_ref, w1_ref, b1_ref, o_ref):
    g = p_ref[...] + q_ref[...] + r_ref[...]
    h = jnp.dot(g, w0_ref[...], preferred_element_type=jnp.float32) + b0_ref[...]
    h = jnp.maximum(h, 0.0)
    o = jnp.dot(h, w1_ref[...], preferred_element_type=jnp.float32) + b1_ref[...]
    o_ref[...] = jnp.maximum(o, 0.0)


ROWS = 1000


@jax.jit
def _mlp(partials, tc_partial, W0, b0, W1, b1):
    grid = (NUM_SEGMENTS // ROWS,)
    return pl.pallas_call(
        _mlp_body,
        grid=grid,
        in_specs=[
            pl.BlockSpec((ROWS, D_FEAT), lambda i: (i, 0)),
            pl.BlockSpec((ROWS, D_FEAT), lambda i: (i + NUM_SEGMENTS // ROWS, 0)),
            pl.BlockSpec((ROWS, D_FEAT), lambda i: (i, 0)),
            pl.BlockSpec((D_FEAT, HIDDEN), lambda i: (0, 0)),
            pl.BlockSpec((1, HIDDEN), lambda i: (0, 0)),
            pl.BlockSpec((HIDDEN, N_OUT), lambda i: (0, 0)),
            pl.BlockSpec((1, N_OUT), lambda i: (0, 0)),
        ],
        out_specs=pl.BlockSpec((ROWS, N_OUT), lambda i: (i, 0)),
        out_shape=jax.ShapeDtypeStruct((NUM_SEGMENTS, N_OUT), jnp.float32),
    )(partials, partials, tc_partial, W0, b0, W1, b1)


def kernel(atom_features, membership, W0, b0, W1, b1):
    mem_i32 = membership.astype(jnp.int32)
    zeros = jnp.zeros((SEG_PER_TILE, D_FEAT), jnp.float32)
    starts = jnp.searchsorted(
        mem_i32, jnp.arange(N_WIN + 1, dtype=jnp.int32) * WIN
    ).astype(jnp.int32)
    starts = jnp.maximum(starts, SC_ATOMS)
    partials = _sc_segsum(atom_features, mem_i32, zeros)
    tc_partial = _tc_win_segsum(atom_features, mem_i32, starts)
    return _mlp(partials, tc_partial, W0, b0.reshape(1, HIDDEN), W1, b1.reshape(1, N_OUT))


# speedup vs baseline: 3.6152x; 1.5575x over previous
import functools

import jax
import jax.numpy as jnp
from jax import lax
from jax.experimental import pallas as pl
from jax.experimental.pallas import tpu as pltpu
from jax.experimental.pallas import tpu_sc as plsc

NUM_SEGMENTS = 10000
N_ATOMS = 320000
D_FEAT = 128
HIDDEN = 256
N_OUT = 128

NC = 2
NS = 16
NW = NC * NS
SC_ATOMS = 266240
PW = SC_ATOMS // NW
CHUNK = 128
N_CHUNKS = PW // CHUNK
TAIL = PW - N_CHUNKS * CHUNK
SEG_PER_TILE = 624
SEG_TAIL = NUM_SEGMENTS - NS * SEG_PER_TILE


def _sc_body(
    atoms_hbm, mem_hbm, zeros_hbm, out_hbm,
    idx0, idx1, abuf0, abuf1, tidx, tbuf, acc,
    semi0, sema0, semi1, sema1, sems0, sems1,
):
    c = lax.axis_index("c")
    s = lax.axis_index("s")
    w = s * NC + c
    row0 = pl.multiple_of(s * SEG_PER_TILE, 8)
    idx = (idx0, idx1)
    abuf = (abuf0, abuf1)
    semi = (semi0, semi1)
    sema = (sema0, sema1)
    sems = (sems0, sems1)

    def start_load(g, slot):
        base = pl.multiple_of(w * PW + g * CHUNK, 8)
        pltpu.async_copy(mem_hbm.at[pl.ds(base, CHUNK)], idx[slot], semi[slot])
        pltpu.async_copy(atoms_hbm.at[pl.ds(base, CHUNK)], abuf[slot], sema[slot])

    def wait_load(slot):
        pltpu.make_async_copy(mem_hbm.at[pl.ds(0, CHUNK)], idx[slot], semi[slot]).wait()
        pltpu.make_async_copy(atoms_hbm.at[pl.ds(0, CHUNK)], abuf[slot], sema[slot]).wait()

    def start_scatter(slot):
        pltpu.async_copy(abuf[slot], acc.at[idx[slot]], sems[slot], add=True)

    def wait_scatter(slot):
        pltpu.make_async_copy(abuf[slot], acc.at[idx[slot]], sems[slot]).wait()

    start_load(0, 0)
    start_load(1, 1)

    pltpu.sync_copy(zeros_hbm, acc.at[pl.ds(row0, SEG_PER_TILE)])

    @pl.when(s == 0)
    def _zero_tail():
        pltpu.sync_copy(
            zeros_hbm.at[pl.ds(0, SEG_TAIL)],
            acc.at[pl.ds(NS * SEG_PER_TILE, SEG_TAIL)],
        )

    plsc.subcore_barrier()

    def step(t, carry):
        g0 = t * 2
        g1 = t * 2 + 1
        wait_load(0)
        pltpu.sync_copy(abuf0, acc.at[idx0], add=True)

        @pl.when(g0 + 2 < N_CHUNKS)
        def _next0():
            start_load(g0 + 2, 0)

        wait_load(1)
        pltpu.sync_copy(abuf1, acc.at[idx1], add=True)

        @pl.when(g1 + 2 < N_CHUNKS)
        def _next1():
            start_load(g1 + 2, 1)

        return carry

    lax.fori_loop(0, N_CHUNKS // 2, step, 0)

    if N_CHUNKS % 2 == 1:
        wait_load(0)
        pltpu.sync_copy(abuf0, acc.at[idx0], add=True)

    if TAIL > 0:
        tbase = pl.multiple_of(w * PW + N_CHUNKS * CHUNK, 8)
        pltpu.sync_copy(mem_hbm.at[pl.ds(tbase, TAIL)], tidx)
        pltpu.sync_copy(atoms_hbm.at[pl.ds(tbase, TAIL)], tbuf)
        pltpu.sync_copy(tbuf, acc.at[tidx], add=True)
    plsc.subcore_barrier()

    orow0 = pl.multiple_of(c * NUM_SEGMENTS + s * SEG_PER_TILE, 8)
    pltpu.sync_copy(
        acc.at[pl.ds(row0, SEG_PER_TILE)],
        out_hbm.at[pl.ds(orow0, SEG_PER_TILE)],
    )

    @pl.when(s == 0)
    def _flush_tail():
        otail = pl.multiple_of(c * NUM_SEGMENTS + NS * SEG_PER_TILE, 8)
        pltpu.sync_copy(
            acc.at[pl.ds(NS * SEG_PER_TILE, SEG_TAIL)],
            out_hbm.at[pl.ds(otail, SEG_TAIL)],
        )


@jax.jit
def _sc_segsum(atoms, mem_i32, zeros):
    mesh = plsc.VectorSubcoreMesh(
        core_axis_name="c", subcore_axis_name="s", num_cores=NC, num_subcores=NS
    )
    f = pl.kernel(
        _sc_body,
        out_type=jax.ShapeDtypeStruct((NC * NUM_SEGMENTS, D_FEAT), jnp.float32),
        mesh=mesh,
        scratch_types=[
            pltpu.VMEM((CHUNK,), jnp.int32),
            pltpu.VMEM((CHUNK,), jnp.int32),
            pltpu.VMEM((CHUNK, D_FEAT), jnp.float32),
            pltpu.VMEM((CHUNK, D_FEAT), jnp.float32),
            pltpu.VMEM((max(TAIL, 8),), jnp.int32),
            pltpu.VMEM((max(TAIL, 8), D_FEAT), jnp.float32),
            pltpu.VMEM_SHARED((NUM_SEGMENTS, D_FEAT), jnp.float32),
            pltpu.SemaphoreType.DMA,
            pltpu.SemaphoreType.DMA,
            pltpu.SemaphoreType.DMA,
            pltpu.SemaphoreType.DMA,
            pltpu.SemaphoreType.DMA,
            pltpu.SemaphoreType.DMA,
        ],
    )
    return f(atoms, mem_i32, zeros)


WIN = 128
N_WIN = 79
TC_BLK = 1024


def _tc_win_body(s_ref, atoms_hbm, mem_hbm, o_ref, abuf, mbuf, sem_a0, sem_m0, sem_a1, sem_m1):
    v = pl.program_id(0)
    lo = s_ref[v]
    hi = s_ref[v + 1]
    dn = (lo // 128) * 128
    nblocks = (hi - dn + TC_BLK - 1) // TC_BLK
    npairs = (nblocks + 1) // 2
    sem_a = (sem_a0, sem_a1)
    sem_m = (sem_m0, sem_m1)

    def blk_base(b):
        base = jnp.minimum(dn + b * TC_BLK, N_ATOMS - TC_BLK)
        return pl.multiple_of(base, 128)

    def start(b, slot):
        base = blk_base(b)
        pltpu.make_async_copy(
            atoms_hbm.at[pl.ds(base, TC_BLK)], abuf.at[slot], sem_a[slot]
        ).start()
        pltpu.make_async_copy(
            mem_hbm.at[pl.ds(base, TC_BLK)], mbuf.at[slot], sem_m[slot]
        ).start()

    def wait(slot):
        pltpu.make_async_copy(
            atoms_hbm.at[pl.ds(0, TC_BLK)], abuf.at[slot], sem_a[slot]
        ).wait()
        pltpu.make_async_copy(
            mem_hbm.at[pl.ds(0, TC_BLK)], mbuf.at[slot], sem_m[slot]
        ).wait()

    def compute(b, slot, acc):
        base = blk_base(b)
        rows = TC_BLK // 128
        gidx = (
            base
            + 128 * jax.lax.broadcasted_iota(jnp.int32, (rows, 128), 0)
            + jax.lax.broadcasted_iota(jnp.int32, (rows, 128), 1)
        )
        valid = (gidx >= jnp.maximum(lo, dn + b * TC_BLK)) & (gidx < hi)
        m2 = mbuf[slot].reshape(rows, 128)
        seg = v * WIN + jax.lax.broadcasted_iota(jnp.int32, (WIN, 1, 1), 0)
        oh = jnp.where(
            (m2[None, :, :] == seg) & valid[None, :, :], 1.0, 0.0
        ).astype(jnp.float32).reshape(WIN, TC_BLK)
        part = jax.lax.dot_general(
            oh, abuf[slot],
            dimension_numbers=(((1,), (0,)), ((), ())),
            preferred_element_type=jnp.float32,
        )
        return acc + part

    @pl.when(nblocks > 0)
    def _p0():
        start(0, 0)

    @pl.when(nblocks > 1)
    def _p1():
        start(1, 1)

    def body(t, acc):
        b0 = t * 2
        b1 = t * 2 + 1
        wait(0)
        acc = compute(b0, 0, acc)

        @pl.when(b0 + 2 < nblocks)
        def _n0():
            start(b0 + 2, 0)

        @pl.when(b1 < nblocks)
        def _w1():
            wait(1)

        acc = compute(b1, 1, acc)

        @pl.when(b1 + 2 < nblocks)
        def _n1():
            start(b1 + 2, 1)

        return acc

    acc0 = jnp.zeros((WIN, D_FEAT), jnp.float32)
    o_ref[...] = lax.fori_loop(0, npairs, body, acc0)


@jax.jit
def _tc_win_segsum(atoms, mem_i32, starts):
    return pl.pallas_call(
        _tc_win_body,
        grid=(N_WIN,),
        in_specs=[
            pl.BlockSpec(memory_space=pltpu.MemorySpace.SMEM),
            pl.BlockSpec(memory_space=pltpu.MemorySpace.HBM),
            pl.BlockSpec(memory_space=pltpu.MemorySpace.HBM),
        ],
        out_specs=pl.BlockSpec((WIN, D_FEAT), lambda v: (v, 0)),
        out_shape=jax.ShapeDtypeStruct((N_WIN * WIN, D_FEAT), jnp.float32),
        scratch_shapes=[
            pltpu.VMEM((2, TC_BLK, D_FEAT), jnp.float32),
            pltpu.VMEM((2, TC_BLK), jnp.int32),
            pltpu.SemaphoreType.DMA,
            pltpu.SemaphoreType.DMA,
            pltpu.SemaphoreType.DMA,
            pltpu.SemaphoreType.DMA,
        ],
    )(starts, atoms, mem_i32)


def _mlp_body(p_ref, q_ref, r_ref, w0_ref, b0_ref, w1_ref, b1_ref, o_ref):
    g = p_ref[...] + q_ref[...] + r_ref[...]
    h = jnp.dot(g, w0_ref[...], preferred_element_type=jnp.float32) + b0_ref[...]
    h = jnp.maximum(h, 0.0)
    o = jnp.dot(h, w1_ref[...], preferred_element_type=jnp.float32) + b1_ref[...]
    o_ref[...] = jnp.maximum(o, 0.0)


ROWS = 2000


@jax.jit
def _mlp(partials, tc_partial, W0, b0, W1, b1):
    grid = (NUM_SEGMENTS // ROWS,)
    return pl.pallas_call(
        _mlp_body,
        grid=grid,
        in_specs=[
            pl.BlockSpec((ROWS, D_FEAT), lambda i: (i, 0)),
            pl.BlockSpec((ROWS, D_FEAT), lambda i: (i + NUM_SEGMENTS // ROWS, 0)),
            pl.BlockSpec((ROWS, D_FEAT), lambda i: (i, 0)),
            pl.BlockSpec((D_FEAT, HIDDEN), lambda i: (0, 0)),
            pl.BlockSpec((1, HIDDEN), lambda i: (0, 0)),
            pl.BlockSpec((HIDDEN, N_OUT), lambda i: (0, 0)),
            pl.BlockSpec((1, N_OUT), lambda i: (0, 0)),
        ],
        out_specs=pl.BlockSpec((ROWS, N_OUT), lambda i: (i, 0)),
        out_shape=jax.ShapeDtypeStruct((NUM_SEGMENTS, N_OUT), jnp.float32),
    )(partials, partials, tc_partial, W0, b0, W1, b1)


def kernel(atom_features, membership, W0, b0, W1, b1):
    mem_i32 = membership.astype(jnp.int32)
    zeros = jnp.zeros((SEG_PER_TILE, D_FEAT), jnp.float32)
    starts = jnp.searchsorted(
        mem_i32, jnp.arange(N_WIN + 1, dtype=jnp.int32) * WIN
    ).astype(jnp.int32)
    starts = jnp.maximum(starts, SC_ATOMS)
    partials = _sc_segsum(atom_features, mem_i32, zeros)
    tc_partial = _tc_win_segsum(atom_features, mem_i32, starts)
    return _mlp(partials, tc_partial, W0, b0.reshape(1, HIDDEN), W1, b1.reshape(1, N_OUT))


# speedup vs baseline: 3.9163x; 1.0833x over previous
import functools

import jax
import jax.numpy as jnp
from jax import lax
from jax.experimental import pallas as pl
from jax.experimental.pallas import tpu as pltpu
from jax.experimental.pallas import tpu_sc as plsc

NUM_SEGMENTS = 10000
N_ATOMS = 320000
D_FEAT = 128
HIDDEN = 256
N_OUT = 128

NC = 2
NS = 16
NW = NC * NS
SC_ATOMS = 266240
PW = SC_ATOMS // NW
CHUNK = 128
N_CHUNKS = PW // CHUNK
TAIL = PW - N_CHUNKS * CHUNK
SEG_PER_TILE = 624
SEG_TAIL = NUM_SEGMENTS - NS * SEG_PER_TILE


def _sc_body(
    atoms_hbm, mem_hbm, zeros_hbm, out_hbm,
    idx0, idx1, abuf0, abuf1, tidx, tbuf, acc,
    semi0, sema0, semi1, sema1, sems0, sems1,
):
    c = lax.axis_index("c")
    s = lax.axis_index("s")
    w = s * NC + c
    row0 = pl.multiple_of(s * SEG_PER_TILE, 8)
    idx = (idx0, idx1)
    abuf = (abuf0, abuf1)
    semi = (semi0, semi1)
    sema = (sema0, sema1)
    sems = (sems0, sems1)

    def start_load(g, slot):
        base = pl.multiple_of(w * PW + g * CHUNK, 8)
        pltpu.async_copy(mem_hbm.at[pl.ds(base, CHUNK)], idx[slot], semi[slot])
        pltpu.async_copy(atoms_hbm.at[pl.ds(base, CHUNK)], abuf[slot], sema[slot])

    def wait_load(slot):
        pltpu.make_async_copy(mem_hbm.at[pl.ds(0, CHUNK)], idx[slot], semi[slot]).wait()
        pltpu.make_async_copy(atoms_hbm.at[pl.ds(0, CHUNK)], abuf[slot], sema[slot]).wait()

    def start_scatter(slot):
        pltpu.async_copy(abuf[slot], acc.at[idx[slot]], sems[slot], add=True)

    def wait_scatter(slot):
        pltpu.make_async_copy(abuf[slot], acc.at[idx[slot]], sems[slot]).wait()

    start_load(0, 0)
    start_load(1, 1)

    pltpu.sync_copy(zeros_hbm, acc.at[pl.ds(row0, SEG_PER_TILE)])

    @pl.when(s == 0)
    def _zero_tail():
        pltpu.sync_copy(
            zeros_hbm.at[pl.ds(0, SEG_TAIL)],
            acc.at[pl.ds(NS * SEG_PER_TILE, SEG_TAIL)],
        )

    plsc.subcore_barrier()

    def step(t, carry):
        g0 = t * 2
        g1 = t * 2 + 1
        wait_load(0)
        pltpu.sync_copy(abuf0, acc.at[idx0], add=True)

        @pl.when(g0 + 2 < N_CHUNKS)
        def _next0():
            start_load(g0 + 2, 0)

        wait_load(1)
        pltpu.sync_copy(abuf1, acc.at[idx1], add=True)

        @pl.when(g1 + 2 < N_CHUNKS)
        def _next1():
            start_load(g1 + 2, 1)

        return carry

    lax.fori_loop(0, N_CHUNKS // 2, step, 0)

    if N_CHUNKS % 2 == 1:
        wait_load(0)
        pltpu.sync_copy(abuf0, acc.at[idx0], add=True)

    if TAIL > 0:
        tbase = pl.multiple_of(w * PW + N_CHUNKS * CHUNK, 8)
        pltpu.sync_copy(mem_hbm.at[pl.ds(tbase, TAIL)], tidx)
        pltpu.sync_copy(atoms_hbm.at[pl.ds(tbase, TAIL)], tbuf)
        pltpu.sync_copy(tbuf, acc.at[tidx], add=True)
    plsc.subcore_barrier()

    orow0 = pl.multiple_of(c * NUM_SEGMENTS + s * SEG_PER_TILE, 8)
    pltpu.sync_copy(
        acc.at[pl.ds(row0, SEG_PER_TILE)],
        out_hbm.at[pl.ds(orow0, SEG_PER_TILE)],
    )

    @pl.when(s == 0)
    def _flush_tail():
        otail = pl.multiple_of(c * NUM_SEGMENTS + NS * SEG_PER_TILE, 8)
        pltpu.sync_copy(
            acc.at[pl.ds(NS * SEG_PER_TILE, SEG_TAIL)],
            out_hbm.at[pl.ds(otail, SEG_TAIL)],
        )


@jax.jit
def _sc_segsum(atoms, mem_i32, zeros):
    mesh = plsc.VectorSubcoreMesh(
        core_axis_name="c", subcore_axis_name="s", num_cores=NC, num_subcores=NS
    )
    f = pl.kernel(
        _sc_body,
        out_type=jax.ShapeDtypeStruct((NC * NUM_SEGMENTS, D_FEAT), jnp.float32),
        mesh=mesh,
        scratch_types=[
            pltpu.VMEM((CHUNK,), jnp.int32),
            pltpu.VMEM((CHUNK,), jnp.int32),
            pltpu.VMEM((CHUNK, D_FEAT), jnp.float32),
            pltpu.VMEM((CHUNK, D_FEAT), jnp.float32),
            pltpu.VMEM((max(TAIL, 8),), jnp.int32),
            pltpu.VMEM((max(TAIL, 8), D_FEAT), jnp.float32),
            pltpu.VMEM_SHARED((NUM_SEGMENTS, D_FEAT), jnp.float32),
            pltpu.SemaphoreType.DMA,
            pltpu.SemaphoreType.DMA,
            pltpu.SemaphoreType.DMA,
            pltpu.SemaphoreType.DMA,
            pltpu.SemaphoreType.DMA,
            pltpu.SemaphoreType.DMA,
        ],
    )
    return f(atoms, mem_i32, zeros)


WIN = 256
N_WIN = 40
TC_BLK = 1024


def _tc_win_body(s_ref, atoms_hbm, mem_hbm, o_ref, abuf, mbuf, sem_a0, sem_m0, sem_a1, sem_m1):
    v = pl.program_id(0)
    lo = s_ref[v]
    hi = s_ref[v + 1]
    dn = (lo // 128) * 128
    nblocks = (hi - dn + TC_BLK - 1) // TC_BLK
    npairs = (nblocks + 1) // 2
    sem_a = (sem_a0, sem_a1)
    sem_m = (sem_m0, sem_m1)

    def blk_base(b):
        base = jnp.minimum(dn + b * TC_BLK, N_ATOMS - TC_BLK)
        return pl.multiple_of(base, 128)

    def start(b, slot):
        base = blk_base(b)
        pltpu.make_async_copy(
            atoms_hbm.at[pl.ds(base, TC_BLK)], abuf.at[slot], sem_a[slot]
        ).start()
        pltpu.make_async_copy(
            mem_hbm.at[pl.ds(base, TC_BLK)], mbuf.at[slot], sem_m[slot]
        ).start()

    def wait(slot):
        pltpu.make_async_copy(
            atoms_hbm.at[pl.ds(0, TC_BLK)], abuf.at[slot], sem_a[slot]
        ).wait()
        pltpu.make_async_copy(
            mem_hbm.at[pl.ds(0, TC_BLK)], mbuf.at[slot], sem_m[slot]
        ).wait()

    def compute(b, slot, acc):
        base = blk_base(b)
        rows = TC_BLK // 128
        gidx = (
            base
            + 128 * jax.lax.broadcasted_iota(jnp.int32, (rows, 128), 0)
            + jax.lax.broadcasted_iota(jnp.int32, (rows, 128), 1)
        )
        valid = (gidx >= jnp.maximum(lo, dn + b * TC_BLK)) & (gidx < hi)
        m2 = mbuf[slot].reshape(rows, 128)
        seg = v * WIN + jax.lax.broadcasted_iota(jnp.int32, (WIN, 1, 1), 0)
        oh = jnp.where(
            (m2[None, :, :] == seg) & valid[None, :, :], 1.0, 0.0
        ).astype(jnp.float32).reshape(WIN, TC_BLK)
        part = jax.lax.dot_general(
            oh, abuf[slot],
            dimension_numbers=(((1,), (0,)), ((), ())),
            preferred_element_type=jnp.float32,
        )
        return acc + part

    @pl.when(nblocks > 0)
    def _p0():
        start(0, 0)

    @pl.when(nblocks > 1)
    def _p1():
        start(1, 1)

    def body(t, acc):
        b0 = t * 2
        b1 = t * 2 + 1
        wait(0)
        acc = compute(b0, 0, acc)

        @pl.when(b0 + 2 < nblocks)
        def _n0():
            start(b0 + 2, 0)

        @pl.when(b1 < nblocks)
        def _w1():
            wait(1)

        acc = compute(b1, 1, acc)

        @pl.when(b1 + 2 < nblocks)
        def _n1():
            start(b1 + 2, 1)

        return acc

    acc0 = jnp.zeros((WIN, D_FEAT), jnp.float32)
    o_ref[...] = lax.fori_loop(0, npairs, body, acc0)


@jax.jit
def _tc_win_segsum(atoms, mem_i32, starts):
    return pl.pallas_call(
        _tc_win_body,
        grid=(N_WIN,),
        in_specs=[
            pl.BlockSpec(memory_space=pltpu.MemorySpace.SMEM),
            pl.BlockSpec(memory_space=pltpu.MemorySpace.HBM),
            pl.BlockSpec(memory_space=pltpu.MemorySpace.HBM),
        ],
        out_specs=pl.BlockSpec((WIN, D_FEAT), lambda v: (v, 0)),
        out_shape=jax.ShapeDtypeStruct((N_WIN * WIN, D_FEAT), jnp.float32),
        scratch_shapes=[
            pltpu.VMEM((2, TC_BLK, D_FEAT), jnp.float32),
            pltpu.VMEM((2, TC_BLK), jnp.int32),
            pltpu.SemaphoreType.DMA,
            pltpu.SemaphoreType.DMA,
            pltpu.SemaphoreType.DMA,
            pltpu.SemaphoreType.DMA,
        ],
    )(starts, atoms, mem_i32)


def _mlp_body(p_ref, q_ref, r_ref, w0_ref, b0_ref, w1_ref, b1_ref, o_ref):
    g = p_ref[...] + q_ref[...] + r_ref[...]
    h = jnp.dot(g, w0_ref[...], preferred_element_type=jnp.float32) + b0_ref[...]
    h = jnp.maximum(h, 0.0)
    o = jnp.dot(h, w1_ref[...], preferred_element_type=jnp.float32) + b1_ref[...]
    o_ref[...] = jnp.maximum(o, 0.0)


ROWS = 10000


@jax.jit
def _mlp(partials, tc_partial, W0, b0, W1, b1):
    grid = (NUM_SEGMENTS // ROWS,)
    return pl.pallas_call(
        _mlp_body,
        grid=grid,
        in_specs=[
            pl.BlockSpec((ROWS, D_FEAT), lambda i: (i, 0)),
            pl.BlockSpec((ROWS, D_FEAT), lambda i: (i + NUM_SEGMENTS // ROWS, 0)),
            pl.BlockSpec((ROWS, D_FEAT), lambda i: (i, 0)),
            pl.BlockSpec((D_FEAT, HIDDEN), lambda i: (0, 0)),
            pl.BlockSpec((1, HIDDEN), lambda i: (0, 0)),
            pl.BlockSpec((HIDDEN, N_OUT), lambda i: (0, 0)),
            pl.BlockSpec((1, N_OUT), lambda i: (0, 0)),
        ],
        out_specs=pl.BlockSpec((ROWS, N_OUT), lambda i: (i, 0)),
        out_shape=jax.ShapeDtypeStruct((NUM_SEGMENTS, N_OUT), jnp.float32),
    )(partials, partials, tc_partial, W0, b0, W1, b1)


def kernel(atom_features, membership, W0, b0, W1, b1):
    mem_i32 = membership.astype(jnp.int32)
    zeros = jnp.zeros((SEG_PER_TILE, D_FEAT), jnp.float32)
    starts = jnp.searchsorted(
        mem_i32, jnp.arange(N_WIN + 1, dtype=jnp.int32) * WIN, method="compare_all"
    ).astype(jnp.int32)
    starts = jnp.maximum(starts, SC_ATOMS)
    partials = _sc_segsum(atom_features, mem_i32, zeros)
    tc_partial = _tc_win_segsum(atom_features, mem_i32, starts)
    return _mlp(partials, tc_partial, W0, b0.reshape(1, HIDDEN), W1, b1.reshape(1, N_OUT))
